# Initial kernel scaffold; baseline (speedup 1.0000x reference)
import functools

import jax
import jax.numpy as jnp
from jax import lax
from jax.experimental import pallas as pl
from jax.experimental.pallas import tpu as pltpu
from jax.experimental.pallas import tpu_sc as plsc

_N = 10000
_E = 160000
_D = 256
_NG = 16
_NC = 10
_BN_S = 1.0 / (1.0 + 1e-5) ** 0.5

_NCORE = 2
_NSUB = 16
_HALF = _N // _NCORE
_ACC_ROWS = 5120
_RPT = _ACC_ROWS // _NSUB
_DUMMY = 5100
_EPT = _E // _NSUB
_CHUNK = 128
_MAXCH = (_EPT + _CHUNK - 1) // _CHUNK


def _agg_body(h_hbm, src_hbm, dst_hbm, out_hbm,
              esrc, edst, srcbuf, dstbuf, stage, zbuf, acc, sem):
    c = lax.axis_index("c")
    s = lax.axis_index("s")

    pltpu.sync_copy(src_hbm.at[pl.ds(s * _EPT, _EPT)], esrc)
    pltpu.sync_copy(dst_hbm.at[pl.ds(s * _EPT, _EPT)], edst)

    zeros_f = jnp.zeros((16,), jnp.float32)
    zeros_i = jnp.zeros((16,), jnp.int32)
    dummy_v = jnp.full((16,), _DUMMY, jnp.int32)

    def zero_body(t, carry):
        zbuf[t >> 4, pl.ds((t & 15) * 16, 16)] = zeros_f
        return carry
    lax.fori_loop(0, 64 * (_D // 16), zero_body, 0)

    def fill_body(t, carry):
        srcbuf[t >> 3, pl.ds((t & 7) * 16, 16)] = zeros_i
        dstbuf[t >> 3, pl.ds((t & 7) * 16, 16)] = dummy_v
        return carry
    lax.fori_loop(0, _MAXCH * (_CHUNK // 16), fill_body, 0)

    for k in range(_RPT // 64):
        pltpu.sync_copy(zbuf, acc.at[pl.ds(s * _RPT + k * 64, 64)])
    plsc.subcore_barrier()

    half_base = c * _HALF

    def comp_body(g, cnt_vec):
        d = edst[pl.ds(g * 16, 16)]
        sr = esrc[pl.ds(g * 16, 16)]
        local = d - half_base
        mask = (local >= 0) & (local < _HALF)
        pos = cnt_vec + plsc.cumsum(jnp.where(mask, 1, 0)) - 1
        row = lax.shift_right_logical(pos, 7)
        col = jnp.bitwise_and(pos, 127)
        plsc.store_scatter(srcbuf, [row, col], sr, mask=mask)
        plsc.store_scatter(dstbuf, [row, col], local, mask=mask)
        return cnt_vec + plsc.all_reduce_population_count(mask)

    cnt_vec = lax.fori_loop(0, _EPT // 16, comp_body,
                            jnp.zeros((16,), jnp.int32))
    nch = lax.shift_right_logical(jnp.max(cnt_vec) + (_CHUNK - 1), 7)

    def gs_body(j, carry):
        pltpu.async_copy(h_hbm.at[srcbuf.at[j]], stage, sem).wait()
        pltpu.sync_copy(stage, acc.at[dstbuf.at[j]], add=True)
        return carry
    lax.fori_loop(0, nch, gs_body, 0)
    plsc.subcore_barrier()

    out_base = c * _HALF + s * _RPT

    @pl.when(s < _NSUB - 1)
    def _copy_full():
        pltpu.sync_copy(acc.at[pl.ds(s * _RPT, _RPT)],
                        out_hbm.at[pl.ds(out_base, _RPT)])

    @pl.when(s == _NSUB - 1)
    def _copy_last():
        last = _HALF - (_NSUB - 1) * _RPT
        pltpu.sync_copy(acc.at[pl.ds(s * _RPT, last)],
                        out_hbm.at[pl.ds(out_base, last)])


_agg_call = functools.partial(
    pl.kernel,
    out_type=jax.ShapeDtypeStruct((_N, _D), jnp.float32),
    mesh=plsc.VectorSubcoreMesh(core_axis_name="c", subcore_axis_name="s"),
    scratch_types=[
        pltpu.VMEM((_EPT,), jnp.int32),
        pltpu.VMEM((_EPT,), jnp.int32),
        pltpu.VMEM((_MAXCH, _CHUNK), jnp.int32),
        pltpu.VMEM((_MAXCH, _CHUNK), jnp.int32),
        pltpu.VMEM((_CHUNK, _D), jnp.float32),
        pltpu.VMEM((64, _D), jnp.float32),
        pltpu.VMEM_SHARED((_ACC_ROWS, _D), jnp.float32),
        pltpu.SemaphoreType.DMA,
    ],
)(_agg_body)


_BLK = 2000


def _mlp_body(eps_ref, h_ref, agg_ref, w1_ref, b1_ref, g1_ref, be1_ref,
              w2_ref, b2_ref, g2_ref, be2_ref, o_ref):
    t = (1.0 + eps_ref[0, 0]) * h_ref[...] + agg_ref[...]
    g1 = g1_ref[...] * _BN_S
    t = jnp.dot(t, w1_ref[...] * g1, preferred_element_type=jnp.float32)
    t = t + (b1_ref[...] * g1 + be1_ref[...])
    t = jnp.where(t > 0, t, 0.01 * t)
    g2 = g2_ref[...] * _BN_S
    t = jnp.dot(t, w2_ref[...] * g2, preferred_element_type=jnp.float32)
    t = t + (b2_ref[...] * g2 + be2_ref[...])
    o_ref[...] = jnp.where(t > 0, t, 0.01 * t)


_row_spec = pl.BlockSpec((_BLK, _D), lambda i: (i, 0))
_w_spec = pl.BlockSpec((_D, _D), lambda i: (0, 0))
_v_spec = pl.BlockSpec((1, _D), lambda i: (0, 0))

_mlp_call = pl.pallas_call(
    _mlp_body,
    grid=(_N // _BLK,),
    in_specs=[
        pl.BlockSpec(memory_space=pltpu.SMEM),
        _row_spec, _row_spec,
        _w_spec, _v_spec, _v_spec, _v_spec,
        _w_spec, _v_spec, _v_spec, _v_spec,
    ],
    out_specs=_row_spec,
    out_shape=jax.ShapeDtypeStruct((_N, _D), jnp.float32),
    compiler_params=pltpu.CompilerParams(
        dimension_semantics=("arbitrary",)),
)


def _pool_body(h_ref, b_ref, wl0_ref, bl0_ref, wlf_ref, blf_ref, o_ref,
               acc_ref, cnt_ref):
    i = pl.program_id(0)

    @pl.when(i == 0)
    def _init():
        acc_ref[...] = jnp.zeros_like(acc_ref)
        cnt_ref[...] = jnp.zeros_like(cnt_ref)

    b = b_ref[0, 0, :]
    oh = (b[None, :] == lax.broadcasted_iota(
        jnp.int32, (_NG, _BLK), 0)).astype(jnp.float32)
    acc_ref[...] += jnp.dot(oh, h_ref[...],
                            preferred_element_type=jnp.float32)
    cnt_ref[...] += jnp.broadcast_to(
        jnp.sum(oh, axis=1, keepdims=True), (_NG, _D))

    @pl.when(i == pl.num_programs(0) - 1)
    def _head():
        xg = acc_ref[...] / jnp.maximum(cnt_ref[...], 1.0)
        xg = jnp.dot(xg, wl0_ref[...],
                     preferred_element_type=jnp.float32) + bl0_ref[...]
        xg = jnp.where(xg > 0, xg, 0.01 * xg)
        o_ref[...] = jnp.dot(xg, wlf_ref[...],
                             preferred_element_type=jnp.float32) + blf_ref[...]


_pool_call = pl.pallas_call(
    _pool_body,
    grid=(_N // _BLK,),
    in_specs=[
        _row_spec,
        pl.BlockSpec((1, 1, _BLK), lambda i: (i, 0, 0)),
        _w_spec, _v_spec,
        pl.BlockSpec((_D, 128), lambda i: (0, 0)),
        pl.BlockSpec((1, 128), lambda i: (0, 0)),
    ],
    out_specs=pl.BlockSpec((_NG, 128), lambda i: (0, 0)),
    out_shape=jax.ShapeDtypeStruct((_NG, 128), jnp.float32),
    scratch_shapes=[
        pltpu.VMEM((_NG, _D), jnp.float32),
        pltpu.VMEM((_NG, _D), jnp.float32),
    ],
    compiler_params=pltpu.CompilerParams(
        dimension_semantics=("arbitrary",)),
)


def kernel(x, edge_index, batch,
           eps0, W1_0, b1_0, g_mlp0, be_mlp0, W2_0, b2_0, g_out0, be_out0,
           eps1, W1_1, b1_1, g_mlp1, be_mlp1, W2_1, b2_1, g_out1, be_out1,
           eps2, W1_2, b1_2, g_mlp2, be_mlp2, W2_2, b2_2, g_out2, be_out2,
           Wl0, bl0, Wlf, blf):
    src = edge_index[0]
    dst = edge_index[1]
    layers = [
        (eps0, W1_0, b1_0, g_mlp0, be_mlp0, W2_0, b2_0, g_out0, be_out0),
        (eps1, W1_1, b1_1, g_mlp1, be_mlp1, W2_1, b2_1, g_out1, be_out1),
        (eps2, W1_2, b1_2, g_mlp2, be_mlp2, W2_2, b2_2, g_out2, be_out2),
    ]
    h = x
    for (eps, w1, b1, g1, be1, w2, b2, g2, be2) in layers:
        agg = _agg_call(h, src, dst)
        h = _mlp_call(eps.reshape(1, 1), h, agg,
                      w1, b1.reshape(1, _D), g1.reshape(1, _D),
                      be1.reshape(1, _D),
                      w2, b2.reshape(1, _D), g2.reshape(1, _D),
                      be2.reshape(1, _D))
    wlf_p = jnp.zeros((_D, 128), jnp.float32).at[:, :_NC].set(Wlf)
    blf_p = jnp.zeros((1, 128), jnp.float32).at[0, :_NC].set(blf)
    xg = _pool_call(h, batch.reshape(_N // _BLK, 1, _BLK),
                    Wl0, bl0.reshape(1, _D), wlf_p, blf_p)
    return (xg[:, :_NC], h)


# speedup vs baseline: 1.0286x; 1.0286x over previous
import functools

import jax
import jax.numpy as jnp
from jax import lax
from jax.experimental import pallas as pl
from jax.experimental.pallas import tpu as pltpu
from jax.experimental.pallas import tpu_sc as plsc

_N = 10000
_E = 160000
_D = 256
_NG = 16
_NC = 10
_BN_S = 1.0 / (1.0 + 1e-5) ** 0.5

_NCORE = 2
_NSUB = 16
_HALF = _N // _NCORE
_ACC_ROWS = 5120
_RPT = _ACC_ROWS // _NSUB
_DUMMY = 5100
_EPT = _E // _NSUB
_ECH = 2000
_CHUNK = 32
_RING = 128


def _agg_body(h_hbm, src_hbm, dst_hbm, out_hbm,
              esrc, edst, srcbuf, dstbuf, sidx, didx, stage, acc, sem):
    c = lax.axis_index("c")
    s = lax.axis_index("s")

    zeros_f = jnp.zeros((16,), jnp.float32)

    def zero_body(t, carry):
        stage[t >> 4, pl.ds((t & 15) * 16, 16)] = zeros_f
        return carry
    lax.fori_loop(0, _CHUNK * (_D // 16), zero_body, 0)

    for k in range(_RPT // _CHUNK):
        pltpu.sync_copy(stage, acc.at[pl.ds(s * _RPT + k * _CHUNK, _CHUNK)])
    plsc.subcore_barrier()

    half_base = c * _HALF

    def drain(j, carry):
        r = jnp.bitwise_and(j, _RING - 1)
        for k in range(_CHUNK // 16):
            sidx[pl.ds(k * 16, 16)] = srcbuf[r, pl.ds(k * 16, 16)]
            didx[pl.ds(k * 16, 16)] = dstbuf[r, pl.ds(k * 16, 16)]
        pltpu.async_copy(h_hbm.at[sidx], stage, sem).wait()
        pltpu.sync_copy(stage, acc.at[didx], add=True)
        return carry

    def chunk_body(ec, carry):
        cnt_vec, dr = carry
        pltpu.sync_copy(src_hbm.at[pl.ds(s * _EPT + ec * _ECH, _ECH)], esrc)
        pltpu.sync_copy(dst_hbm.at[pl.ds(s * _EPT + ec * _ECH, _ECH)], edst)

        def comp_body(g, cnt_vec):
            d = edst[pl.ds(g * 16, 16)]
            sr = esrc[pl.ds(g * 16, 16)]
            local = d - half_base
            mask = (local >= 0) & (local < _HALF)
            pos = cnt_vec + plsc.cumsum(jnp.where(mask, 1, 0)) - 1
            row = jnp.bitwise_and(lax.shift_right_logical(pos, 5), _RING - 1)
            col = jnp.bitwise_and(pos, _CHUNK - 1)
            plsc.store_scatter(srcbuf, [row, col], sr, mask=mask)
            plsc.store_scatter(dstbuf, [row, col], local, mask=mask)
            return cnt_vec + plsc.all_reduce_population_count(mask)

        cnt_vec = lax.fori_loop(0, _ECH // 16, comp_body, cnt_vec)
        target = lax.shift_right_logical(jnp.max(cnt_vec), 5)
        lax.fori_loop(dr, target, drain, 0)
        return (cnt_vec, target)

    cnt_vec, dr = lax.fori_loop(
        0, _EPT // _ECH, chunk_body,
        (jnp.zeros((16,), jnp.int32), jnp.int32(0)))

    cnt = jnp.max(cnt_vec)
    iota16 = lax.broadcasted_iota(jnp.int32, (16,), 0)
    for k in range(_CHUNK // 16):
        p = cnt + iota16 + k * 16
        row = jnp.bitwise_and(lax.shift_right_logical(p, 5), _RING - 1)
        col = jnp.bitwise_and(p, _CHUNK - 1)
        plsc.store_scatter(srcbuf, [row, col], jnp.zeros((16,), jnp.int32))
        plsc.store_scatter(dstbuf, [row, col],
                           jnp.full((16,), _DUMMY, jnp.int32))
    target = lax.shift_right_logical(cnt + (_CHUNK - 1), 5)
    lax.fori_loop(dr, target, drain, 0)
    plsc.subcore_barrier()

    out_base = c * _HALF + s * _RPT

    @pl.when(s < _NSUB - 1)
    def _copy_full():
        pltpu.sync_copy(acc.at[pl.ds(s * _RPT, _RPT)],
                        out_hbm.at[pl.ds(out_base, _RPT)])

    @pl.when(s == _NSUB - 1)
    def _copy_last():
        last = _HALF - (_NSUB - 1) * _RPT
        pltpu.sync_copy(acc.at[pl.ds(s * _RPT, last)],
                        out_hbm.at[pl.ds(out_base, last)])


@functools.cache
def _get_agg_call():
    return functools.partial(
        pl.kernel,
        out_type=jax.ShapeDtypeStruct((_N, _D), jnp.float32),
        mesh=plsc.VectorSubcoreMesh(core_axis_name="c", subcore_axis_name="s",
                                    num_cores=_NCORE, num_subcores=_NSUB),
        compiler_params=pltpu.CompilerParams(needs_layout_passes=False),
        scratch_types=[
            pltpu.VMEM((_ECH,), jnp.int32),
            pltpu.VMEM((_ECH,), jnp.int32),
            pltpu.VMEM((_RING, _CHUNK), jnp.int32),
            pltpu.VMEM((_RING, _CHUNK), jnp.int32),
            pltpu.VMEM((_CHUNK,), jnp.int32),
            pltpu.VMEM((_CHUNK,), jnp.int32),
            pltpu.VMEM((_CHUNK, _D), jnp.float32),
            pltpu.VMEM_SHARED((_ACC_ROWS, _D), jnp.float32),
            pltpu.SemaphoreType.DMA,
        ],
    )(_agg_body)


_BLK = 2000


def _mlp_body(eps_ref, h_ref, agg_ref, w1_ref, b1_ref, g1_ref, be1_ref,
              w2_ref, b2_ref, g2_ref, be2_ref, o_ref):
    t = (1.0 + eps_ref[0, 0]) * h_ref[...] + agg_ref[...]
    g1 = g1_ref[...] * _BN_S
    t = jnp.dot(t, w1_ref[...] * g1, preferred_element_type=jnp.float32)
    t = t + (b1_ref[...] * g1 + be1_ref[...])
    t = jnp.where(t > 0, t, 0.01 * t)
    g2 = g2_ref[...] * _BN_S
    t = jnp.dot(t, w2_ref[...] * g2, preferred_element_type=jnp.float32)
    t = t + (b2_ref[...] * g2 + be2_ref[...])
    o_ref[...] = jnp.where(t > 0, t, 0.01 * t)


_row_spec = pl.BlockSpec((_BLK, _D), lambda i: (i, 0))
_w_spec = pl.BlockSpec((_D, _D), lambda i: (0, 0))
_v_spec = pl.BlockSpec((1, _D), lambda i: (0, 0))

_mlp_call = pl.pallas_call(
    _mlp_body,
    grid=(_N // _BLK,),
    in_specs=[
        pl.BlockSpec(memory_space=pltpu.SMEM),
        _row_spec, _row_spec,
        _w_spec, _v_spec, _v_spec, _v_spec,
        _w_spec, _v_spec, _v_spec, _v_spec,
    ],
    out_specs=_row_spec,
    out_shape=jax.ShapeDtypeStruct((_N, _D), jnp.float32),
    compiler_params=pltpu.CompilerParams(
        dimension_semantics=("arbitrary",)),
)


def _pool_body(h_ref, b_ref, wl0_ref, bl0_ref, wlf_ref, blf_ref, o_ref,
               acc_ref, cnt_ref):
    i = pl.program_id(0)

    @pl.when(i == 0)
    def _init():
        acc_ref[...] = jnp.zeros_like(acc_ref)
        cnt_ref[...] = jnp.zeros_like(cnt_ref)

    b = b_ref[0, 0, :]
    oh = (b[None, :] == lax.broadcasted_iota(
        jnp.int32, (_NG, _BLK), 0)).astype(jnp.float32)
    acc_ref[...] += jnp.dot(oh, h_ref[...],
                            preferred_element_type=jnp.float32)
    cnt_ref[...] += jnp.broadcast_to(
        jnp.sum(oh, axis=1, keepdims=True), (_NG, _D))

    @pl.when(i == pl.num_programs(0) - 1)
    def _head():
        xg = acc_ref[...] / jnp.maximum(cnt_ref[...], 1.0)
        xg = jnp.dot(xg, wl0_ref[...],
                     preferred_element_type=jnp.float32) + bl0_ref[...]
        xg = jnp.where(xg > 0, xg, 0.01 * xg)
        o_ref[...] = jnp.dot(xg, wlf_ref[...],
                             preferred_element_type=jnp.float32) + blf_ref[...]


_pool_call = pl.pallas_call(
    _pool_body,
    grid=(_N // _BLK,),
    in_specs=[
        _row_spec,
        pl.BlockSpec((1, 1, _BLK), lambda i: (i, 0, 0)),
        _w_spec, _v_spec,
        pl.BlockSpec((_D, 128), lambda i: (0, 0)),
        pl.BlockSpec((1, 128), lambda i: (0, 0)),
    ],
    out_specs=pl.BlockSpec((_NG, 128), lambda i: (0, 0)),
    out_shape=jax.ShapeDtypeStruct((_NG, 128), jnp.float32),
    scratch_shapes=[
        pltpu.VMEM((_NG, _D), jnp.float32),
        pltpu.VMEM((_NG, _D), jnp.float32),
    ],
    compiler_params=pltpu.CompilerParams(
        dimension_semantics=("arbitrary",)),
)


def kernel(x, edge_index, batch,
           eps0, W1_0, b1_0, g_mlp0, be_mlp0, W2_0, b2_0, g_out0, be_out0,
           eps1, W1_1, b1_1, g_mlp1, be_mlp1, W2_1, b2_1, g_out1, be_out1,
           eps2, W1_2, b1_2, g_mlp2, be_mlp2, W2_2, b2_2, g_out2, be_out2,
           Wl0, bl0, Wlf, blf):
    src = edge_index[0]
    dst = edge_index[1]
    layers = [
        (eps0, W1_0, b1_0, g_mlp0, be_mlp0, W2_0, b2_0, g_out0, be_out0),
        (eps1, W1_1, b1_1, g_mlp1, be_mlp1, W2_1, b2_1, g_out1, be_out1),
        (eps2, W1_2, b1_2, g_mlp2, be_mlp2, W2_2, b2_2, g_out2, be_out2),
    ]
    h = x
    for (eps, w1, b1, g1, be1, w2, b2, g2, be2) in layers:
        agg = jax.ops.segment_sum(h[src], dst, num_segments=_N)
        h = _mlp_call(eps.reshape(1, 1), h, agg,
                      w1, b1.reshape(1, _D), g1.reshape(1, _D),
                      be1.reshape(1, _D),
                      w2, b2.reshape(1, _D), g2.reshape(1, _D),
                      be2.reshape(1, _D))
    wlf_p = jnp.zeros((_D, 128), jnp.float32).at[:, :_NC].set(Wlf)
    blf_p = jnp.zeros((1, 128), jnp.float32).at[0, :_NC].set(blf)
    xg = _pool_call(h, batch.reshape(_N // _BLK, 1, _BLK),
                    Wl0, bl0.reshape(1, _D), wlf_p, blf_p)
    return (xg[:, :_NC], h)


# speedup vs baseline: 1.4564x; 1.4159x over previous
import functools

import jax
import jax.numpy as jnp
from jax import lax
from jax.experimental import pallas as pl
from jax.experimental.pallas import tpu as pltpu
from jax.experimental.pallas import tpu_sc as plsc

_N = 10000
_E = 160000
_D = 256
_NG = 16
_NC = 10
_BN_S = 1.0 / (1.0 + 1e-5) ** 0.5

_NCORE = 2
_NSUB = 16
_NW = _NCORE * _NSUB
_OWN = 313
_OWN_LAST = _N - (_NW - 1) * _OWN
_ACCR = 320
_DUMMY = 316
_ECH = 2000
_CHUNK = 32
_RING = 128


def _agg_body(h_hbm, src_hbm, dst_hbm, out_hbm,
              esrc, edst, srcbuf, dstbuf, sidx, stage, acc, sem):
    c = lax.axis_index("c")
    s = lax.axis_index("s")
    wid = c * _NSUB + s
    base = wid * _OWN
    own_n = jnp.minimum(_OWN, _N - base)

    zeros_f = jnp.zeros((16,), jnp.float32)
    iota16 = lax.broadcasted_iota(jnp.int32, (16,), 0)

    def zacc(t, carry):
        acc[pl.ds(t * 16, 16)] = zeros_f
        return carry
    lax.fori_loop(0, _ACCR * (_D // 16), zacc, 0)

    def drain(j, carry):
        r = jnp.bitwise_and(j, _RING - 1)
        for k in range(_CHUNK // 16):
            sidx[pl.ds(k * 16, 16)] = srcbuf[r, pl.ds(k * 16, 16)]
        pltpu.async_copy(h_hbm.at[sidx], stage, sem).wait()
        for k16 in range(_CHUNK // 16):
            dvec = dstbuf[r, pl.ds(k16 * 16, 16)]
            for k in range(16):
                base_v = dvec.at[jnp.full((16,), k, jnp.int32)].get(
                    mode="promise_in_bounds")
                for f in range(_D // 16):
                    plsc.addupdate_scatter(
                        acc, [base_v + (f * 16) + iota16],
                        stage[k16 * 16 + k, pl.ds(f * 16, 16)])
        return carry

    def chunk_body(ec, carry):
        cnt_vec, dr = carry
        pltpu.sync_copy(src_hbm.at[pl.ds(ec * _ECH, _ECH)], esrc)
        pltpu.sync_copy(dst_hbm.at[pl.ds(ec * _ECH, _ECH)], edst)

        def comp_body(g, cnt_vec):
            d = edst[pl.ds(g * 16, 16)]
            sr = esrc[pl.ds(g * 16, 16)]
            local = d - base
            mask = (local >= 0) & (local < own_n)
            pos = cnt_vec + plsc.cumsum(jnp.where(mask, 1, 0)) - 1
            row = jnp.bitwise_and(lax.shift_right_logical(pos, 5), _RING - 1)
            col = jnp.bitwise_and(pos, _CHUNK - 1)
            plsc.store_scatter(srcbuf, [row, col], sr, mask=mask)
            plsc.store_scatter(dstbuf, [row, col], local * _D, mask=mask)
            return cnt_vec + plsc.all_reduce_population_count(mask)

        cnt_vec = lax.fori_loop(0, _ECH // 16, comp_body, cnt_vec)
        target = lax.shift_right_logical(jnp.max(cnt_vec), 5)
        lax.fori_loop(dr, target, drain, 0)
        return (cnt_vec, target)

    cnt_vec, dr = lax.fori_loop(
        0, _E // _ECH, chunk_body,
        (jnp.zeros((16,), jnp.int32), jnp.int32(0)))

    cnt = jnp.max(cnt_vec)
    for k in range(_CHUNK // 16):
        p = cnt + iota16 + k * 16
        row = jnp.bitwise_and(lax.shift_right_logical(p, 5), _RING - 1)
        col = jnp.bitwise_and(p, _CHUNK - 1)
        plsc.store_scatter(srcbuf, [row, col], jnp.zeros((16,), jnp.int32))
        plsc.store_scatter(dstbuf, [row, col],
                           jnp.full((16,), _DUMMY * _D, jnp.int32))
    target = lax.shift_right_logical(cnt + (_CHUNK - 1), 5)
    lax.fori_loop(dr, target, drain, 0)

    @pl.when(wid < _NW - 1)
    def _copy_full():
        pltpu.sync_copy(acc.at[pl.ds(0, _OWN * _D)],
                        out_hbm.at[pl.ds(base * _D, _OWN * _D)])

    @pl.when(wid == _NW - 1)
    def _copy_last():
        pltpu.sync_copy(acc.at[pl.ds(0, _OWN_LAST * _D)],
                        out_hbm.at[pl.ds(base * _D, _OWN_LAST * _D)])


@functools.cache
def _get_agg_call():
    return functools.partial(
        pl.kernel,
        out_type=jax.ShapeDtypeStruct((_N * _D,), jnp.float32),
        mesh=plsc.VectorSubcoreMesh(core_axis_name="c", subcore_axis_name="s",
                                    num_cores=_NCORE, num_subcores=_NSUB),
        compiler_params=pltpu.CompilerParams(needs_layout_passes=False),
        scratch_types=[
            pltpu.VMEM((_ECH,), jnp.int32),
            pltpu.VMEM((_ECH,), jnp.int32),
            pltpu.VMEM((_RING, _CHUNK), jnp.int32),
            pltpu.VMEM((_RING, _CHUNK), jnp.int32),
            pltpu.VMEM((_CHUNK,), jnp.int32),
            pltpu.VMEM((_CHUNK, _D), jnp.float32),
            pltpu.VMEM((_ACCR * _D,), jnp.float32),
            pltpu.SemaphoreType.DMA,
        ],
    )(_agg_body)


_BLK = 2000


def _mlp_body(eps_ref, h_ref, agg_ref, w1_ref, b1_ref, g1_ref, be1_ref,
              w2_ref, b2_ref, g2_ref, be2_ref, o_ref):
    t = (1.0 + eps_ref[0, 0]) * h_ref[...] + agg_ref[...]
    g1 = g1_ref[...] * _BN_S
    t = jnp.dot(t, w1_ref[...] * g1, preferred_element_type=jnp.float32)
    t = t + (b1_ref[...] * g1 + be1_ref[...])
    t = jnp.where(t > 0, t, 0.01 * t)
    g2 = g2_ref[...] * _BN_S
    t = jnp.dot(t, w2_ref[...] * g2, preferred_element_type=jnp.float32)
    t = t + (b2_ref[...] * g2 + be2_ref[...])
    o_ref[...] = jnp.where(t > 0, t, 0.01 * t)


_row_spec = pl.BlockSpec((_BLK, _D), lambda i: (i, 0))
_w_spec = pl.BlockSpec((_D, _D), lambda i: (0, 0))
_v_spec = pl.BlockSpec((1, _D), lambda i: (0, 0))

_mlp_call = pl.pallas_call(
    _mlp_body,
    grid=(_N // _BLK,),
    in_specs=[
        pl.BlockSpec(memory_space=pltpu.SMEM),
        _row_spec, _row_spec,
        _w_spec, _v_spec, _v_spec, _v_spec,
        _w_spec, _v_spec, _v_spec, _v_spec,
    ],
    out_specs=_row_spec,
    out_shape=jax.ShapeDtypeStruct((_N, _D), jnp.float32),
    compiler_params=pltpu.CompilerParams(
        dimension_semantics=("arbitrary",)),
)


def _pool_body(h_ref, b_ref, wl0_ref, bl0_ref, wlf_ref, blf_ref, o_ref,
               acc_ref, cnt_ref):
    i = pl.program_id(0)

    @pl.when(i == 0)
    def _init():
        acc_ref[...] = jnp.zeros_like(acc_ref)
        cnt_ref[...] = jnp.zeros_like(cnt_ref)

    b = b_ref[0, 0, :]
    oh = (b[None, :] == lax.broadcasted_iota(
        jnp.int32, (_NG, _BLK), 0)).astype(jnp.float32)
    acc_ref[...] += jnp.dot(oh, h_ref[...],
                            preferred_element_type=jnp.float32)
    cnt_ref[...] += jnp.broadcast_to(
        jnp.sum(oh, axis=1, keepdims=True), (_NG, _D))

    @pl.when(i == pl.num_programs(0) - 1)
    def _head():
        xg = acc_ref[...] / jnp.maximum(cnt_ref[...], 1.0)
        xg = jnp.dot(xg, wl0_ref[...],
                     preferred_element_type=jnp.float32) + bl0_ref[...]
        xg = jnp.where(xg > 0, xg, 0.01 * xg)
        o_ref[...] = jnp.dot(xg, wlf_ref[...],
                             preferred_element_type=jnp.float32) + blf_ref[...]


_pool_call = pl.pallas_call(
    _pool_body,
    grid=(_N // _BLK,),
    in_specs=[
        _row_spec,
        pl.BlockSpec((1, 1, _BLK), lambda i: (i, 0, 0)),
        _w_spec, _v_spec,
        pl.BlockSpec((_D, 128), lambda i: (0, 0)),
        pl.BlockSpec((1, 128), lambda i: (0, 0)),
    ],
    out_specs=pl.BlockSpec((_NG, 128), lambda i: (0, 0)),
    out_shape=jax.ShapeDtypeStruct((_NG, 128), jnp.float32),
    scratch_shapes=[
        pltpu.VMEM((_NG, _D), jnp.float32),
        pltpu.VMEM((_NG, _D), jnp.float32),
    ],
    compiler_params=pltpu.CompilerParams(
        dimension_semantics=("arbitrary",)),
)


def kernel(x, edge_index, batch,
           eps0, W1_0, b1_0, g_mlp0, be_mlp0, W2_0, b2_0, g_out0, be_out0,
           eps1, W1_1, b1_1, g_mlp1, be_mlp1, W2_1, b2_1, g_out1, be_out1,
           eps2, W1_2, b1_2, g_mlp2, be_mlp2, W2_2, b2_2, g_out2, be_out2,
           Wl0, bl0, Wlf, blf):
    src = edge_index[0]
    dst = edge_index[1]
    layers = [
        (eps0, W1_0, b1_0, g_mlp0, be_mlp0, W2_0, b2_0, g_out0, be_out0),
        (eps1, W1_1, b1_1, g_mlp1, be_mlp1, W2_1, b2_1, g_out1, be_out1),
        (eps2, W1_2, b1_2, g_mlp2, be_mlp2, W2_2, b2_2, g_out2, be_out2),
    ]
    agg_call = _get_agg_call()
    h = x
    for (eps, w1, b1, g1, be1, w2, b2, g2, be2) in layers:
        agg = agg_call(h, src, dst).reshape(_N, _D)
        h = _mlp_call(eps.reshape(1, 1), h, agg,
                      w1, b1.reshape(1, _D), g1.reshape(1, _D),
                      be1.reshape(1, _D),
                      w2, b2.reshape(1, _D), g2.reshape(1, _D),
                      be2.reshape(1, _D))
    wlf_p = jnp.zeros((_D, 128), jnp.float32).at[:, :_NC].set(Wlf)
    blf_p = jnp.zeros((1, 128), jnp.float32).at[0, :_NC].set(blf)
    xg = _pool_call(h, batch.reshape(_N // _BLK, 1, _BLK),
                    Wl0, bl0.reshape(1, _D), wlf_p, blf_p)
    return (xg[:, :_NC], h)
